# Initial kernel scaffold; baseline (speedup 1.0000x reference)
import jax
import jax.numpy as jnp
from jax.experimental import pallas as pl


def kernel(detect, segment, img_size, nc):
    raise NotImplementedError("write your pallas kernel here")


# speedup vs baseline: 3.0150x; 3.0150x over previous
import functools

import jax
import jax.numpy as jnp
import numpy as np
from jax import lax
from jax.experimental import pallas as pl

_NM = 32
_NC = 80
_NBOX = 1000
_NPAD = 1024
_NSLOT = 128
_MAXDET = 100
_MH = 128
_IMG = 512
_ROWT = 16
_NEG = -1e30


def _interp_matrix() -> np.ndarray:
    pos = (np.arange(_IMG, dtype=np.float64) + 0.5) * (_MH / _IMG) - 0.5
    lo = np.floor(pos).astype(np.int64)
    frac = pos - lo
    m = np.zeros((_IMG, _MH), dtype=np.float64)
    np.add.at(m, (np.arange(_IMG), np.clip(lo, 0, _MH - 1)), 1.0 - frac)
    np.add.at(m, (np.arange(_IMG), np.clip(lo + 1, 0, _MH - 1)), frac)
    return m.astype(np.float32)


_RMAT = _interp_matrix()


def _nms_body(boxes_ref, cls_ref, selidx_ref, selcls_ref, selkeep_ref,
              bx1_ref, by1_ref, bx2_ref, by2_ref):
    cls = cls_ref[...]
    maxp = jnp.max(cls, axis=0, keepdims=True)
    rio = lax.broadcasted_iota(jnp.int32, (_NC, _NPAD), 0)
    cid = jnp.min(jnp.where(cls == maxp, rio, 2 ** 30), axis=0,
                  keepdims=True).astype(jnp.float32)
    coli = lax.broadcasted_iota(jnp.int32, (1, _NPAD), 1)
    col = coli.astype(jnp.float32)
    valid = (maxp >= 0.4) & (coli < _NBOX)
    score0 = jnp.where(valid, maxp, _NEG)
    x1 = boxes_ref[0:1, :]
    y1 = boxes_ref[1:2, :]
    x2 = boxes_ref[2:3, :]
    y2 = boxes_ref[3:4, :]
    area = (x2 - x1) * (y2 - y1)

    selidx_ref[...] = jnp.full((_NSLOT, 1), -1.0, jnp.float32)
    selcls_ref[...] = jnp.full((_NSLOT, 1), -1.0, jnp.float32)
    selkeep_ref[...] = jnp.zeros((_NSLOT, 1), jnp.float32)
    bx1_ref[...] = jnp.zeros((_NSLOT, 1), jnp.float32)
    by1_ref[...] = jnp.zeros((_NSLOT, 1), jnp.float32)
    bx2_ref[...] = jnp.zeros((_NSLOT, 1), jnp.float32)
    by2_ref[...] = jnp.zeros((_NSLOT, 1), jnp.float32)

    def body(t, score):
        maxv = jnp.max(score)
        widx = jnp.min(jnp.where(score >= maxv, col, 1e9))
        kflag = maxv > -1e29
        ohf = (col == widx).astype(jnp.float32)
        wx1 = jnp.sum(x1 * ohf)
        wy1 = jnp.sum(y1 * ohf)
        wx2 = jnp.sum(x2 * ohf)
        wy2 = jnp.sum(y2 * ohf)
        wcid = jnp.sum(cid * ohf)
        warea = (wx2 - wx1) * (wy2 - wy1)
        ix1 = jnp.maximum(x1, wx1)
        iy1 = jnp.maximum(y1, wy1)
        ix2 = jnp.minimum(x2, wx2)
        iy2 = jnp.minimum(y2, wy2)
        inter = (ix2 - ix1) * (iy2 - iy1)
        iou = inter / (area + warea - inter)
        sup = (iou >= 0.7) & kflag
        nscore = jnp.where(sup | (col == widx), _NEG, score)
        kf = kflag.astype(jnp.float32)
        selidx_ref[pl.ds(t, 1), :] = jnp.where(kflag, widx, -1.0).reshape(1, 1)
        selcls_ref[pl.ds(t, 1), :] = wcid.reshape(1, 1)
        selkeep_ref[pl.ds(t, 1), :] = kf.reshape(1, 1)
        bx1_ref[pl.ds(t, 1), :] = wx1.reshape(1, 1)
        by1_ref[pl.ds(t, 1), :] = wy1.reshape(1, 1)
        bx2_ref[pl.ds(t, 1), :] = wx2.reshape(1, 1)
        by2_ref[pl.ds(t, 1), :] = wy2.reshape(1, 1)
        return nscore

    lax.fori_loop(0, _MAXDET, body, score0)


def _sig_body(coef_ref, proto_ref, selidx_ref, selcls_ref, selkeep_ref,
              sig_ref, ohc_ref):
    col = lax.broadcasted_iota(jnp.int32, (1, _NPAD), 1).astype(jnp.float32)
    ohsel = (selidx_ref[...] == col).astype(jnp.float32)
    selcoef = lax.dot_general(ohsel, coef_ref[...],
                              (((1,), (1,)), ((), ())),
                              precision=lax.Precision.HIGHEST,
                              preferred_element_type=jnp.float32)
    logits = lax.dot_general(selcoef, proto_ref[...],
                             (((1,), (0,)), ((), ())),
                             preferred_element_type=jnp.float32)
    sig_ref[...] = jax.nn.sigmoid(logits)
    li = lax.broadcasted_iota(jnp.int32, (1, _NSLOT), 1).astype(jnp.float32)
    ohc_ref[...] = (selcls_ref[...] == li).astype(jnp.float32) * selkeep_ref[...]


def _comp_body(sig_ref, ohc_ref, bx1_ref, by1_ref, bx2_ref, by2_ref,
               r_ref, c_ref, out_ref):
    i = pl.program_id(0)
    sig3 = sig_ref[...].reshape(_NSLOT, _MH, _MH)
    d1 = lax.dot_general(sig3, r_ref[...],
                         (((1,), (1,)), ((), ())),
                         precision=lax.Precision.HIGHEST,
                         preferred_element_type=jnp.float32)
    up = lax.dot_general(d1, c_ref[...],
                         (((1,), (1,)), ((), ())),
                         precision=lax.Precision.HIGHEST,
                         preferred_element_type=jnp.float32)
    rowf = (i * _ROWT).astype(jnp.float32) + lax.broadcasted_iota(
        jnp.int32, (1, _ROWT, 1), 1).astype(jnp.float32)
    colf = lax.broadcasted_iota(jnp.int32, (1, 1, _IMG), 2).astype(jnp.float32)
    x1 = bx1_ref[...].reshape(_NSLOT, 1, 1)
    y1 = by1_ref[...].reshape(_NSLOT, 1, 1)
    x2 = bx2_ref[...].reshape(_NSLOT, 1, 1)
    y2 = by2_ref[...].reshape(_NSLOT, 1, 1)
    inbox = (colf >= x1) & (colf < x2) & (rowf >= y1) & (rowf < y2)
    m = jnp.where((up > 0.5) & inbox, up, 0.0)
    m2 = m.reshape(_NSLOT, _ROWT * _IMG)
    o = lax.dot_general(m2, ohc_ref[...],
                        (((0,), (0,)), ((), ())),
                        preferred_element_type=jnp.float32)
    out_ref[...] = o[:, :_NC].reshape(_ROWT, _IMG, _NC)


def kernel(detect, segment, img_size, nc):
    del img_size, nc
    det = jnp.pad(detect.astype(jnp.float32), ((0, 0), (0, _NPAD - _NBOX)))
    boxes = det[0:4, :]
    cls = det[4:4 + _NC, :]
    coef = det[4 + _NC:, :]
    proto = segment.astype(jnp.float32).reshape(_NM, _MH * _MH)
    rmat = jnp.asarray(_RMAT)

    col1 = jax.ShapeDtypeStruct((_NSLOT, 1), jnp.float32)
    selidx, selcls, selkeep, bx1, by1, bx2, by2 = pl.pallas_call(
        _nms_body,
        out_shape=(col1, col1, col1, col1, col1, col1, col1),
    )(boxes, cls)

    sig, ohc = pl.pallas_call(
        _sig_body,
        out_shape=(
            jax.ShapeDtypeStruct((_NSLOT, _MH * _MH), jnp.float32),
            jax.ShapeDtypeStruct((_NSLOT, _NSLOT), jnp.float32),
        ),
    )(coef, proto, selidx, selcls, selkeep)

    nsteps = _IMG // _ROWT
    full = lambda s: pl.BlockSpec(s, lambda i: tuple(0 for _ in s))
    out = pl.pallas_call(
        _comp_body,
        grid=(nsteps,),
        in_specs=[
            full((_NSLOT, _MH * _MH)),
            full((_NSLOT, _NSLOT)),
            full((_NSLOT, 1)),
            full((_NSLOT, 1)),
            full((_NSLOT, 1)),
            full((_NSLOT, 1)),
            pl.BlockSpec((_ROWT, _MH), lambda i: (i, 0)),
            full((_IMG, _MH)),
        ],
        out_specs=pl.BlockSpec((_ROWT, _IMG, _NC), lambda i: (i, 0, 0)),
        out_shape=jax.ShapeDtypeStruct((_IMG, _IMG, _NC), jnp.float32),
    )(sig, ohc, bx1, by1, bx2, by2, rmat, rmat)

    return out[None]


# speedup vs baseline: 3.8709x; 1.2839x over previous
import functools

import jax
import jax.numpy as jnp
import numpy as np
from jax import lax
from jax.experimental import pallas as pl

_NM = 32
_NC = 80
_NBOX = 1000
_NSLOT = 128
_MAXDET = 100
_MH = 128
_IMG = 512
_ROWT = 32
_NEG = -1e30


def _interp_matrix() -> np.ndarray:
    pos = (np.arange(_IMG, dtype=np.float64) + 0.5) * (_MH / _IMG) - 0.5
    lo = np.floor(pos).astype(np.int64)
    frac = pos - lo
    m = np.zeros((_IMG, _MH), dtype=np.float64)
    np.add.at(m, (np.arange(_IMG), np.clip(lo, 0, _MH - 1)), 1.0 - frac)
    np.add.at(m, (np.arange(_IMG), np.clip(lo + 1, 0, _MH - 1)), frac)
    return m.astype(np.float32)


_RMAT = _interp_matrix()


def _nms_body(det_ref, selidx_ref, selcls_ref, selkeep_ref,
              bx1_ref, by1_ref, bx2_ref, by2_ref):
    cls = det_ref[4:4 + _NC, :]
    maxp = jnp.max(cls, axis=0, keepdims=True)
    rio = lax.broadcasted_iota(jnp.int32, (_NC, _NBOX), 0)
    cid = jnp.min(jnp.where(cls == maxp, rio, 2 ** 30), axis=0,
                  keepdims=True).astype(jnp.float32)
    coli = lax.broadcasted_iota(jnp.int32, (1, _NBOX), 1)
    col = coli.astype(jnp.float32)
    score0 = jnp.where(maxp >= 0.4, maxp, _NEG)
    x1 = det_ref[0:1, :]
    y1 = det_ref[1:2, :]
    x2 = det_ref[2:3, :]
    y2 = det_ref[3:4, :]
    area = (x2 - x1) * (y2 - y1)

    selidx_ref[...] = jnp.full((_NSLOT, 1), -1.0, jnp.float32)
    selcls_ref[...] = jnp.full((_NSLOT, 1), -1.0, jnp.float32)
    selkeep_ref[...] = jnp.zeros((_NSLOT, 1), jnp.float32)
    bx1_ref[...] = jnp.zeros((_NSLOT, 1), jnp.float32)
    by1_ref[...] = jnp.zeros((_NSLOT, 1), jnp.float32)
    bx2_ref[...] = jnp.zeros((_NSLOT, 1), jnp.float32)
    by2_ref[...] = jnp.zeros((_NSLOT, 1), jnp.float32)

    def body(t, score):
        maxv = jnp.max(score)
        widx = jnp.min(jnp.where(score >= maxv, col, 1e9))
        kflag = maxv > -1e29
        ohf = (col == widx).astype(jnp.float32)
        wx1 = jnp.sum(x1 * ohf)
        wy1 = jnp.sum(y1 * ohf)
        wx2 = jnp.sum(x2 * ohf)
        wy2 = jnp.sum(y2 * ohf)
        wcid = jnp.sum(cid * ohf)
        warea = (wx2 - wx1) * (wy2 - wy1)
        ix1 = jnp.maximum(x1, wx1)
        iy1 = jnp.maximum(y1, wy1)
        ix2 = jnp.minimum(x2, wx2)
        iy2 = jnp.minimum(y2, wy2)
        inter = (ix2 - ix1) * (iy2 - iy1)
        iou = inter / (area + warea - inter)
        sup = (iou >= 0.7) & kflag
        nscore = jnp.where(sup | (col == widx), _NEG, score)
        kf = kflag.astype(jnp.float32)
        selidx_ref[pl.ds(t, 1), :] = jnp.where(kflag, widx, -1.0).reshape(1, 1)
        selcls_ref[pl.ds(t, 1), :] = wcid.reshape(1, 1)
        selkeep_ref[pl.ds(t, 1), :] = kf.reshape(1, 1)
        bx1_ref[pl.ds(t, 1), :] = wx1.reshape(1, 1)
        by1_ref[pl.ds(t, 1), :] = wy1.reshape(1, 1)
        bx2_ref[pl.ds(t, 1), :] = wx2.reshape(1, 1)
        by2_ref[pl.ds(t, 1), :] = wy2.reshape(1, 1)
        return nscore

    lax.fori_loop(0, _MAXDET, body, score0)


def _sig_body(det_ref, proto_ref, selidx_ref, selcls_ref, selkeep_ref,
              sig_ref, ohc_ref):
    col = lax.broadcasted_iota(jnp.int32, (1, _NBOX), 1).astype(jnp.float32)
    ohsel = (selidx_ref[...] == col).astype(jnp.float32)
    selcoef = lax.dot_general(ohsel, det_ref[4 + _NC:, :],
                              (((1,), (1,)), ((), ())),
                              precision=lax.Precision.HIGHEST,
                              preferred_element_type=jnp.float32)
    logits = lax.dot_general(selcoef, proto_ref[...],
                             (((1,), (0,)), ((), ())),
                             preferred_element_type=jnp.float32)
    sig_ref[...] = jax.nn.sigmoid(logits)
    li = lax.broadcasted_iota(jnp.int32, (1, _NSLOT), 1).astype(jnp.float32)
    ohc_ref[...] = (selcls_ref[...] == li).astype(jnp.float32) * selkeep_ref[...]


def _comp_body(sig_ref, ohc_ref, bx1_ref, by1_ref, bx2_ref, by2_ref,
               r_ref, c_ref, out_ref):
    i = pl.program_id(0)
    sig3 = sig_ref[...].reshape(_NSLOT, _MH, _MH)
    d1 = lax.dot_general(sig3, r_ref[...],
                         (((1,), (1,)), ((), ())),
                         precision=lax.Precision.HIGHEST,
                         preferred_element_type=jnp.float32)
    up = lax.dot_general(d1, c_ref[...],
                         (((1,), (1,)), ((), ())),
                         precision=lax.Precision.HIGHEST,
                         preferred_element_type=jnp.float32)
    rowf = (i * _ROWT).astype(jnp.float32) + lax.broadcasted_iota(
        jnp.int32, (1, _ROWT, 1), 1).astype(jnp.float32)
    colf = lax.broadcasted_iota(jnp.int32, (1, 1, _IMG), 2).astype(jnp.float32)
    x1 = bx1_ref[...].reshape(_NSLOT, 1, 1)
    y1 = by1_ref[...].reshape(_NSLOT, 1, 1)
    x2 = bx2_ref[...].reshape(_NSLOT, 1, 1)
    y2 = by2_ref[...].reshape(_NSLOT, 1, 1)
    inbox = (colf >= x1) & (colf < x2) & (rowf >= y1) & (rowf < y2)
    m = jnp.where((up > 0.5) & inbox, up, 0.0)
    m2 = m.reshape(_NSLOT, _ROWT * _IMG)
    o = lax.dot_general(m2, ohc_ref[...],
                        (((0,), (0,)), ((), ())),
                        preferred_element_type=jnp.float32)
    out_ref[...] = o[:, :_NC].reshape(_ROWT, _IMG, _NC)


def kernel(detect, segment, img_size, nc):
    del img_size, nc
    det = detect.astype(jnp.float32)
    proto = segment.astype(jnp.float32).reshape(_NM, _MH * _MH)
    rmat = jnp.asarray(_RMAT)

    col1 = jax.ShapeDtypeStruct((_NSLOT, 1), jnp.float32)
    selidx, selcls, selkeep, bx1, by1, bx2, by2 = pl.pallas_call(
        _nms_body,
        out_shape=(col1, col1, col1, col1, col1, col1, col1),
    )(det)

    sig, ohc = pl.pallas_call(
        _sig_body,
        out_shape=(
            jax.ShapeDtypeStruct((_NSLOT, _MH * _MH), jnp.float32),
            jax.ShapeDtypeStruct((_NSLOT, _NSLOT), jnp.float32),
        ),
    )(det, proto, selidx, selcls, selkeep)

    nsteps = _IMG // _ROWT
    full = lambda s: pl.BlockSpec(s, lambda i: tuple(0 for _ in s))
    out = pl.pallas_call(
        _comp_body,
        grid=(nsteps,),
        in_specs=[
            full((_NSLOT, _MH * _MH)),
            full((_NSLOT, _NSLOT)),
            full((_NSLOT, 1)),
            full((_NSLOT, 1)),
            full((_NSLOT, 1)),
            full((_NSLOT, 1)),
            pl.BlockSpec((_ROWT, _MH), lambda i: (i, 0)),
            full((_IMG, _MH)),
        ],
        out_specs=pl.BlockSpec((_ROWT, _IMG, _NC), lambda i: (i, 0, 0)),
        out_shape=jax.ShapeDtypeStruct((_IMG, _IMG, _NC), jnp.float32),
    )(sig, ohc, bx1, by1, bx2, by2, rmat, rmat)

    return out[None]


# speedup vs baseline: 7.5592x; 1.9528x over previous
import functools

import jax
import jax.numpy as jnp
import numpy as np
from jax import lax
from jax.experimental import pallas as pl
from jax.experimental.pallas import tpu as pltpu

_NM = 32
_NC = 80
_NBOX = 1000
_NSLOT = 128
_MAXDET = 100
_MH = 128
_IMG = 512
_SUP = 128
_ROWT = 32
_NEG = -1e30


def _interp_matrix() -> np.ndarray:
    pos = (np.arange(_IMG, dtype=np.float64) + 0.5) * (_MH / _IMG) - 0.5
    lo = np.floor(pos).astype(np.int64)
    frac = pos - lo
    m = np.zeros((_IMG, _MH), dtype=np.float64)
    np.add.at(m, (np.arange(_IMG), np.clip(lo, 0, _MH - 1)), 1.0 - frac)
    np.add.at(m, (np.arange(_IMG), np.clip(lo + 1, 0, _MH - 1)), frac)
    return m.astype(np.float32)


_RMAT = _interp_matrix()


def _nms_body(det_ref, selidx_ref, selcls_ref, selkeep_ref,
              bx1_ref, by1_ref, bx2_ref, by2_ref):
    cls = det_ref[4:4 + _NC, :]
    maxp = jnp.max(cls, axis=0, keepdims=True)
    rio = lax.broadcasted_iota(jnp.int32, (_NC, _NBOX), 0)
    cid = jnp.min(jnp.where(cls == maxp, rio, 2 ** 30), axis=0,
                  keepdims=True).astype(jnp.float32)
    coli = lax.broadcasted_iota(jnp.int32, (1, _NBOX), 1)
    col = coli.astype(jnp.float32)
    score0 = jnp.where(maxp >= 0.4, maxp, _NEG)
    x1 = det_ref[0:1, :]
    y1 = det_ref[1:2, :]
    x2 = det_ref[2:3, :]
    y2 = det_ref[3:4, :]
    area = (x2 - x1) * (y2 - y1)

    selidx_ref[...] = jnp.full((_NSLOT, 1), -1.0, jnp.float32)
    selcls_ref[...] = jnp.full((_NSLOT, 1), -1.0, jnp.float32)
    selkeep_ref[...] = jnp.zeros((_NSLOT, 1), jnp.float32)
    bx1_ref[...] = jnp.zeros((_NSLOT, 1), jnp.float32)
    by1_ref[...] = jnp.zeros((_NSLOT, 1), jnp.float32)
    bx2_ref[...] = jnp.zeros((_NSLOT, 1), jnp.float32)
    by2_ref[...] = jnp.zeros((_NSLOT, 1), jnp.float32)

    def body(t, score):
        maxv = jnp.max(score)
        widx = jnp.min(jnp.where(score >= maxv, col, 1e9))
        kflag = maxv > -1e29
        ohf = (col == widx).astype(jnp.float32)
        wx1 = jnp.sum(x1 * ohf)
        wy1 = jnp.sum(y1 * ohf)
        wx2 = jnp.sum(x2 * ohf)
        wy2 = jnp.sum(y2 * ohf)
        wcid = jnp.sum(cid * ohf)
        warea = (wx2 - wx1) * (wy2 - wy1)
        ix1 = jnp.maximum(x1, wx1)
        iy1 = jnp.maximum(y1, wy1)
        ix2 = jnp.minimum(x2, wx2)
        iy2 = jnp.minimum(y2, wy2)
        inter = (ix2 - ix1) * (iy2 - iy1)
        iou = inter / (area + warea - inter)
        sup = (iou >= 0.7) & kflag
        nscore = jnp.where(sup | (col == widx), _NEG, score)
        kf = kflag.astype(jnp.float32)
        selidx_ref[pl.ds(t, 1), :] = jnp.where(kflag, widx, -1.0).reshape(1, 1)
        selcls_ref[pl.ds(t, 1), :] = wcid.reshape(1, 1)
        selkeep_ref[pl.ds(t, 1), :] = kf.reshape(1, 1)
        bx1_ref[pl.ds(t, 1), :] = wx1.reshape(1, 1)
        by1_ref[pl.ds(t, 1), :] = wy1.reshape(1, 1)
        bx2_ref[pl.ds(t, 1), :] = wx2.reshape(1, 1)
        by2_ref[pl.ds(t, 1), :] = wy2.reshape(1, 1)
        return nscore

    lax.fori_loop(0, _MAXDET, body, score0)


def _sig_body(det_ref, proto_ref, selidx_ref, selcls_ref, selkeep_ref,
              sig_ref, ohc_ref):
    col = lax.broadcasted_iota(jnp.int32, (1, _NBOX), 1).astype(jnp.float32)
    ohsel = (selidx_ref[...] == col).astype(jnp.float32)
    selcoef = lax.dot_general(ohsel, det_ref[4 + _NC:, :],
                              (((1,), (1,)), ((), ())),
                              precision=lax.Precision.HIGHEST,
                              preferred_element_type=jnp.float32)
    logits = lax.dot_general(selcoef, proto_ref[...],
                             (((1,), (0,)), ((), ())),
                             preferred_element_type=jnp.float32)
    sig_ref[...] = jax.nn.sigmoid(logits)
    li = lax.broadcasted_iota(jnp.int32, (1, _NSLOT), 1).astype(jnp.float32)
    ohc_ref[...] = (selcls_ref[...] == li).astype(jnp.float32) * selkeep_ref[...]


def _comp_body(sig_ref, ohc_ref, bx1_ref, by1_ref, bx2_ref, by2_ref,
               r_ref, c_ref, out_ref, d1_ref):
    o = pl.program_id(0)
    j = pl.program_id(1)

    @pl.when(j == 0)
    def _():
        sig3 = sig_ref[...].reshape(_NSLOT, _MH, _MH)
        d1_ref[...] = lax.dot_general(r_ref[...], sig3,
                                      (((1,), (1,)), ((), ())),
                                      precision=lax.Precision.HIGHEST,
                                      preferred_element_type=jnp.float32)

    d1s = d1_ref[pl.ds(j * _ROWT, _ROWT), :, :]
    up = lax.dot_general(d1s, c_ref[...],
                         (((2,), (1,)), ((), ())),
                         precision=lax.Precision.HIGHEST,
                         preferred_element_type=jnp.float32)
    rowf = (o * _SUP + j * _ROWT).astype(jnp.float32) + lax.broadcasted_iota(
        jnp.int32, (_ROWT, 1, 1), 0).astype(jnp.float32)
    colf = lax.broadcasted_iota(jnp.int32, (1, 1, _IMG), 2).astype(jnp.float32)
    x1 = bx1_ref[...].reshape(1, _NSLOT, 1)
    y1 = by1_ref[...].reshape(1, _NSLOT, 1)
    x2 = bx2_ref[...].reshape(1, _NSLOT, 1)
    y2 = by2_ref[...].reshape(1, _NSLOT, 1)
    inbox = (colf >= x1) & (colf < x2) & (rowf >= y1) & (rowf < y2)
    m = jnp.where((up > 0.5) & inbox, up, 0.0)
    comp = lax.dot_general(m, ohc_ref[...],
                           (((1,), (0,)), ((), ())),
                           preferred_element_type=jnp.float32)
    out_ref[...] = comp[:, :, :_NC]


def kernel(detect, segment, img_size, nc):
    del img_size, nc
    det = detect.astype(jnp.float32)
    proto = segment.astype(jnp.float32).reshape(_NM, _MH * _MH)
    rmat = jnp.asarray(_RMAT)

    col1 = jax.ShapeDtypeStruct((_NSLOT, 1), jnp.float32)
    selidx, selcls, selkeep, bx1, by1, bx2, by2 = pl.pallas_call(
        _nms_body,
        out_shape=(col1, col1, col1, col1, col1, col1, col1),
    )(det)

    sig, ohc = pl.pallas_call(
        _sig_body,
        out_shape=(
            jax.ShapeDtypeStruct((_NSLOT, _MH * _MH), jnp.float32),
            jax.ShapeDtypeStruct((_NSLOT, _NSLOT), jnp.float32),
        ),
    )(det, proto, selidx, selcls, selkeep)

    nsup = _IMG // _SUP
    nsub = _SUP // _ROWT
    full = lambda s: pl.BlockSpec(s, lambda o, j: tuple(0 for _ in s))
    out = pl.pallas_call(
        _comp_body,
        grid=(nsup, nsub),
        in_specs=[
            full((_NSLOT, _MH * _MH)),
            full((_NSLOT, _NSLOT)),
            full((_NSLOT, 1)),
            full((_NSLOT, 1)),
            full((_NSLOT, 1)),
            full((_NSLOT, 1)),
            pl.BlockSpec((_SUP, _MH), lambda o, j: (o, 0)),
            full((_IMG, _MH)),
        ],
        out_specs=pl.BlockSpec((_ROWT, _IMG, _NC), lambda o, j: (o * nsub + j, 0, 0)),
        out_shape=jax.ShapeDtypeStruct((_IMG, _IMG, _NC), jnp.float32),
        scratch_shapes=[pltpu.VMEM((_SUP, _NSLOT, _MH), jnp.float32)],
    )(sig, ohc, bx1, by1, bx2, by2, rmat, rmat)

    return out[None]
